# baseline (speedup 1.0000x reference)
import functools

import jax
import jax.numpy as jnp
from jax import lax
from jax.experimental import pallas as pl
from jax.experimental.pallas import tpu as pltpu
from jax.experimental.pallas import tpu_sc as plsc

NODES = 1000000
D = 64
BB = 16384
FF = 26
TOTAL = BB * FF

_info = plsc.get_sparse_core_info()
_NC = _info.num_cores
_NW = _NC * _info.num_subcores

_NBLK = NODES // 128
_TRIPS = _NBLK // _NW
_LEFT = _NBLK - _TRIPS * _NW

_mesh = plsc.VectorSubcoreMesh(core_axis_name="c", subcore_axis_name="s")


def _transpose_block(inref, outref, n_srows, riota):
    @plsc.parallel_loop(0, n_srows, unroll=4)
    def _srow(s):
        ce = jnp.full((16,), 2 * s, jnp.int32)
        co = jnp.full((16,), 2 * s + 1, jnp.int32)
        vs = [
            plsc.load_gather(inref, [riota[kg % 4], ce if kg < 4 else co])
            for kg in range(8)
        ]
        for kg in range(8):
            outref[s, pl.ds(kg * 16, 16)] = vs[kg]


@functools.partial(
    pl.kernel,
    mesh=_mesh,
    out_type=jax.ShapeDtypeStruct((NODES // 2, 128), jnp.float32),
    scratch_types=(
        [pltpu.VMEM((64, 128), jnp.float32) for _ in range(4)]
        + [pltpu.SemaphoreType.DMA for _ in range(4)]
    ),
    compiler_params=pltpu.CompilerParams(
        needs_layout_passes=False, disable_bounds_checks=True
    ),
)
def _fmt_kernel(wt_hbm, tail_hbm, stag_hbm, inb0, inb1, outb0, outb1, is0, is1, os0, os1):
    wid = lax.axis_index("s") * _NC + lax.axis_index("c")
    inb = (inb0, inb1)
    outb = (outb0, outb1)
    isem = (is0, is1)
    osem = (os0, os1)
    riota = [lax.iota(jnp.int32, 16) + 16 * kg for kg in range(4)]

    def rt_of(i):
        return wid + _NW * i

    def fire_in(i, par):
        pltpu.async_copy(
            wt_hbm.at[:, pl.ds(rt_of(i) * 128, 128)], inb[par], isem[par]
        )

    def drain(sem, par, buf):
        pltpu.make_async_copy(
            wt_hbm.at[:, pl.ds(0, 128)], buf[par], sem[par]
        ).wait()

    fire_in(0, 0)
    fire_in(1, 1)

    def pair(g, carry):
        for par in range(2):
            i = 2 * g + par
            drain(isem, par, inb)

            @pl.when(g >= 1)
            def _():
                drain(osem, par, outb)

            _transpose_block(inb[par], outb[par], 64, riota)
            pltpu.async_copy(
                outb[par], stag_hbm.at[pl.ds(rt_of(i) * 64, 64)], osem[par]
            )
            fire_in(i + 2, par)
        return carry

    lax.fori_loop(0, _TRIPS // 2 - 1, pair, 0)

    for par in range(2):
        i = _TRIPS - 2 + par
        drain(isem, par, inb)
        drain(osem, par, outb)
        _transpose_block(inb[par], outb[par], 64, riota)
        pltpu.async_copy(
            outb[par], stag_hbm.at[pl.ds(rt_of(i) * 64, 64)], osem[par]
        )
    for par in range(2):
        drain(osem, par, outb)

    for k in range(_LEFT):

        @pl.when(wid == k)
        def _():
            rt = _TRIPS * _NW + k
            pltpu.sync_copy(wt_hbm.at[:, pl.ds(rt * 128, 128)], inb0)
            _transpose_block(inb0, outb0, 64, riota)
            pltpu.sync_copy(outb0, stag_hbm.at[pl.ds(rt * 64, 64)])

    @pl.when(wid == _LEFT)
    def _():
        base = (_TRIPS * _NW + _LEFT) * 128
        pltpu.sync_copy(tail_hbm, inb0.at[pl.ds(0, 32)])
        pltpu.sync_copy(inb0.at[pl.ds(0, 32)], stag_hbm.at[pl.ds(base // 2, 32)])


_PW = TOTAL // _NW
_BC = 16
_CR = _BC * FF
_NCH = _PW // _CR


@functools.partial(
    pl.kernel,
    mesh=_mesh,
    out_type=jax.ShapeDtypeStruct((FF, 8, BB // 128, 8, 128), jnp.float32),
    scratch_types=(
        [pltpu.VMEM((_PW,), jnp.int32)]
        + [pltpu.VMEM((_CR, D), jnp.float32) for _ in range(2)]
        + [pltpu.VMEM((FF, 8, 8, _BC), jnp.float32) for _ in range(2)]
        + [pltpu.SemaphoreType.DMA for _ in range(4)]
    ),
    compiler_params=pltpu.CompilerParams(
        use_tc_tiling_on_sc=False,
        needs_layout_passes=False,
        disable_bounds_checks=True,
    ),
)
def _gather_kernel(idx_hbm, stag_hbm, out_hbm, idx_v, r0, r1, v0, v1, g0, g1, w0, w1):
    wid = lax.axis_index("s") * _NC + lax.axis_index("c")
    base = wid * _PW
    rows = (r0, r1)
    vbuf = (v0, v1)
    gsem = (g0, g1)
    wsem = (w0, w1)
    i26 = lax.iota(jnp.int32, 16) * FF

    pltpu.sync_copy(idx_hbm.at[pl.ds(base, _PW)], idx_v)

    def fire_gathers(c, par):
        off = c * _CR
        for j, (o, n) in enumerate(((0, 128), (128, 128), (256, 128), (384, 32))):
            pltpu.async_copy(
                stag_hbm.at[idx_v.at[pl.ds(off + o, n)]],
                rows[par].at[pl.ds(o, n)],
                gsem[par],
            )

    def drain_gathers(par):
        for o, n in ((0, 128), (128, 128), (256, 128), (384, 32)):
            pltpu.make_async_copy(
                stag_hbm.at[idx_v.at[pl.ds(0, n)]],
                rows[par].at[pl.ds(o, n)],
                gsem[par],
            ).wait()

    def drain_write(par):
        pltpu.make_async_copy(
            out_hbm.at[:, :, 0, :, pl.ds(0, _BC)], vbuf[par], wsem[par]
        ).wait()

    def transpose_chunk(par):
        @plsc.parallel_loop(0, FF, unroll=2)
        def _fbody(f):
            rowi = i26 + f
            for jt in range(8):
                vs = [
                    plsc.load_gather(
                        rows[par],
                        [rowi, jnp.full((16,), jt * 8 + j, jnp.int32)],
                    )
                    for j in range(8)
                ]
                for j in range(8):
                    vbuf[par][f, jt, j, :] = vs[j]

    fire_gathers(0, 0)
    fire_gathers(1, 1)

    def visit(g, carry):
        for par in range(2):
            c = 2 * g + par
            drain_gathers(par)

            @pl.when(c >= 2)
            def _():
                drain_write(par)

            transpose_chunk(par)
            b0 = wid * 512 + c * _BC
            bt = b0 // 128
            bo = b0 % 128
            pltpu.async_copy(
                vbuf[par],
                out_hbm.at[:, :, bt, :, pl.ds(bo, _BC)],
                wsem[par],
            )
            cn = jnp.minimum(c + 2, _NCH - 1)
            fire_gathers(cn, par)
        return carry

    lax.fori_loop(0, _NCH // 2 - 1, visit, 0)

    for par in range(2):
        c = _NCH - 2 + par
        drain_gathers(par)
        drain_write(par)
        transpose_chunk(par)
        b0 = wid * 512 + c * _BC
        pltpu.async_copy(
            vbuf[par],
            out_hbm.at[:, :, b0 // 128, :, pl.ds(b0 % 128, _BC)],
            wsem[par],
        )
    for par in range(2):
        drain_write(par)


def kernel(x, m, W):
    idx = x.reshape(TOTAL).astype(jnp.int32)
    tail = W[_NBLK * 128:].reshape(32, 128)
    stag = _fmt_kernel(W.T, tail).reshape(NODES, D)
    o5 = _gather_kernel(idx, stag)
    return o5.transpose(2, 4, 0, 1, 3).reshape(BB, FF, D)


# speedup vs baseline: 1.1732x; 1.1732x over previous
import functools

import jax
import jax.numpy as jnp
from jax import lax
from jax.experimental import pallas as pl
from jax.experimental.pallas import tpu as pltpu
from jax.experimental.pallas import tpu_sc as plsc

NODES = 1000000
D = 64
BB = 16384
FF = 26
TOTAL = BB * FF

_info = plsc.get_sparse_core_info()
_NC = _info.num_cores
_NW = _NC * _info.num_subcores

_NBLK = NODES // 128
_TRIPS = _NBLK // _NW
_LEFT = _NBLK - _TRIPS * _NW

_mesh = plsc.VectorSubcoreMesh(core_axis_name="c", subcore_axis_name="s")


def _transpose_block(inref, outref, n_srows, rr, par64):
    ngr = n_srows // 8

    @plsc.parallel_loop(0, 64, unroll=2)
    def _feat(j):
        cj = jnp.full((16,), j, jnp.int32) + par64
        for rg in range(ngr):
            v = inref[j, pl.ds(rg * 16, 16)]
            plsc.store_scatter(outref, [rr[rg], cj], v)


@functools.partial(
    pl.kernel,
    mesh=_mesh,
    out_type=jax.ShapeDtypeStruct((NODES // 2, 128), jnp.float32),
    scratch_types=(
        [pltpu.VMEM((64, 128), jnp.float32) for _ in range(4)]
        + [pltpu.SemaphoreType.DMA for _ in range(4)]
    ),
    compiler_params=pltpu.CompilerParams(
        needs_layout_passes=False, disable_bounds_checks=True
    ),
)
def _fmt_kernel(wt_hbm, tail_hbm, stag_hbm, inb0, inb1, outb0, outb1, is0, is1, os0, os1):
    wid = lax.axis_index("s") * _NC + lax.axis_index("c")
    inb = (inb0, inb1)
    outb = (outb0, outb1)
    isem = (is0, is1)
    osem = (os0, os1)
    iot = lax.iota(jnp.int32, 16)
    rr = [(iot >> 1) + 8 * rg for rg in range(8)]
    par64 = (iot & 1) * 64

    def rt_of(i):
        return wid + _NW * i

    def fire_in(i, par):
        pltpu.async_copy(
            wt_hbm.at[:, pl.ds(rt_of(i) * 128, 128)], inb[par], isem[par]
        )

    def drain(sem, par, buf):
        pltpu.make_async_copy(
            wt_hbm.at[:, pl.ds(0, 128)], buf[par], sem[par]
        ).wait()

    fire_in(0, 0)
    fire_in(1, 1)

    def pair(g, carry):
        for par in range(2):
            i = 2 * g + par
            drain(isem, par, inb)

            @pl.when(g >= 1)
            def _():
                drain(osem, par, outb)

            _transpose_block(inb[par], outb[par], 64, rr, par64)
            pltpu.async_copy(
                outb[par], stag_hbm.at[pl.ds(rt_of(i) * 64, 64)], osem[par]
            )
            fire_in(i + 2, par)
        return carry

    lax.fori_loop(0, _TRIPS // 2 - 1, pair, 0)

    for par in range(2):
        i = _TRIPS - 2 + par
        drain(isem, par, inb)
        drain(osem, par, outb)
        _transpose_block(inb[par], outb[par], 64, rr, par64)
        pltpu.async_copy(
            outb[par], stag_hbm.at[pl.ds(rt_of(i) * 64, 64)], osem[par]
        )
    for par in range(2):
        drain(osem, par, outb)

    for k in range(_LEFT):

        @pl.when(wid == k)
        def _():
            rt = _TRIPS * _NW + k
            pltpu.sync_copy(wt_hbm.at[:, pl.ds(rt * 128, 128)], inb0)
            _transpose_block(inb0, outb0, 64, rr, par64)
            pltpu.sync_copy(outb0, stag_hbm.at[pl.ds(rt * 64, 64)])

    @pl.when(wid == _LEFT)
    def _():
        base = (_TRIPS * _NW + _LEFT) * 128
        pltpu.sync_copy(tail_hbm, inb0.at[pl.ds(0, 32)])
        pltpu.sync_copy(inb0.at[pl.ds(0, 32)], stag_hbm.at[pl.ds(base // 2, 32)])


_PW = TOTAL // _NW
_BC = 16
_CR = _BC * FF
_NCH = _PW // _CR


@functools.partial(
    pl.kernel,
    mesh=_mesh,
    out_type=jax.ShapeDtypeStruct((FF, 8, BB // 128, 8, 128), jnp.float32),
    scratch_types=(
        [pltpu.VMEM((_PW,), jnp.int32)]
        + [pltpu.VMEM((_CR, D), jnp.float32) for _ in range(2)]
        + [pltpu.VMEM((FF, 8, 8, _BC), jnp.float32) for _ in range(2)]
        + [pltpu.SemaphoreType.DMA for _ in range(4)]
    ),
    compiler_params=pltpu.CompilerParams(
        use_tc_tiling_on_sc=False,
        needs_layout_passes=False,
        disable_bounds_checks=True,
    ),
)
def _gather_kernel(idx_hbm, stag_hbm, out_hbm, idx_v, r0, r1, v0, v1, g0, g1, w0, w1):
    wid = lax.axis_index("s") * _NC + lax.axis_index("c")
    base = wid * _PW
    rows = (r0, r1)
    vbuf = (v0, v1)
    gsem = (g0, g1)
    wsem = (w0, w1)
    iot = lax.iota(jnp.int32, 16)

    pltpu.sync_copy(idx_hbm.at[pl.ds(base, _PW)], idx_v)

    def fire_gathers(c, par):
        off = c * _CR
        for j, (o, n) in enumerate(((0, 128), (128, 128), (256, 128), (384, 32))):
            pltpu.async_copy(
                stag_hbm.at[idx_v.at[pl.ds(off + o, n)]],
                rows[par].at[pl.ds(o, n)],
                gsem[par],
            )

    def drain_gathers(par):
        for o, n in ((0, 128), (128, 128), (256, 128), (384, 32)):
            pltpu.make_async_copy(
                stag_hbm.at[idx_v.at[pl.ds(0, n)]],
                rows[par].at[pl.ds(o, n)],
                gsem[par],
            ).wait()

    def drain_write(par):
        pltpu.make_async_copy(
            out_hbm.at[:, :, 0, :, pl.ds(0, _BC)], vbuf[par], wsem[par]
        ).wait()

    kt = [(iot >> 3) + 2 * kg for kg in range(4)]
    k7 = iot & 7
    bs = [jnp.full((16,), b, jnp.int32) for b in range(_BC)]

    def transpose_chunk(par):
        @plsc.parallel_loop(0, FF, unroll=2)
        def _fbody(f):
            fs = jnp.full((16,), f, jnp.int32)
            for b in range(_BC):
                for kg in range(4):
                    v = rows[par][b * FF + f, pl.ds(kg * 16, 16)]
                    plsc.store_scatter(vbuf[par], [fs, kt[kg], k7, bs[b]], v)

    fire_gathers(0, 0)
    fire_gathers(1, 1)

    def visit(g, carry):
        for par in range(2):
            c = 2 * g + par
            drain_gathers(par)

            @pl.when(c >= 2)
            def _():
                drain_write(par)

            transpose_chunk(par)
            b0 = wid * 512 + c * _BC
            bt = b0 // 128
            bo = b0 % 128
            pltpu.async_copy(
                vbuf[par],
                out_hbm.at[:, :, bt, :, pl.ds(bo, _BC)],
                wsem[par],
            )
            cn = jnp.minimum(c + 2, _NCH - 1)
            fire_gathers(cn, par)
        return carry

    lax.fori_loop(0, _NCH // 2 - 1, visit, 0)

    for par in range(2):
        c = _NCH - 2 + par
        drain_gathers(par)
        drain_write(par)
        transpose_chunk(par)
        b0 = wid * 512 + c * _BC
        pltpu.async_copy(
            vbuf[par],
            out_hbm.at[:, :, b0 // 128, :, pl.ds(b0 % 128, _BC)],
            wsem[par],
        )
    for par in range(2):
        drain_write(par)


def kernel(x, m, W):
    idx = x.reshape(TOTAL).astype(jnp.int32)
    tail = W[_NBLK * 128:].reshape(32, 128)
    stag = _fmt_kernel(W.T, tail).reshape(NODES, D)
    o5 = _gather_kernel(idx, stag)
    return o5.transpose(2, 4, 0, 1, 3).reshape(BB, FF, D)


# speedup vs baseline: 1.4222x; 1.2122x over previous
import functools

import jax
import jax.numpy as jnp
from jax import lax
from jax.experimental import pallas as pl
from jax.experimental.pallas import tpu as pltpu
from jax.experimental.pallas import tpu_sc as plsc

_EMB_DIM = 64
_G = 128
_NSLOT = 8


@functools.lru_cache(maxsize=None)
def _build(total: int):
    info = plsc.get_sparse_core_info()
    nw = info.num_cores * info.num_subcores
    per_w = total // nw
    n_chunks = per_w // _G
    n_rounds = n_chunks // _NSLOT
    assert per_w % _G == 0 and n_chunks % _NSLOT == 0 and n_rounds >= 2

    mesh = plsc.VectorSubcoreMesh(core_axis_name="c", subcore_axis_name="s")

    @functools.partial(
        pl.kernel,
        mesh=mesh,
        out_type=jax.ShapeDtypeStruct((total, _EMB_DIM), jnp.float32),
        scratch_types=(
            [pltpu.VMEM((per_w,), jnp.int32)]
            + [pltpu.VMEM((_G, _EMB_DIM), jnp.float32) for _ in range(_NSLOT)]
            + [pltpu.SemaphoreType.DMA for _ in range(_NSLOT)]
        ),
        compiler_params=pltpu.CompilerParams(use_tc_tiling_on_sc=False),
    )
    def gather_kernel(idx_hbm, table_hbm, out_hbm, idx_v, *rest):
        slots = rest[:_NSLOT]
        gsems = rest[_NSLOT:]
        wid = lax.axis_index("s") * info.num_cores + lax.axis_index("c")
        base = wid * per_w

        pltpu.sync_copy(idx_hbm.at[pl.ds(base, per_w)], idx_v)

        def fire(c, s):
            pltpu.async_copy(
                table_hbm.at[idx_v.at[pl.ds(c * _G, _G)]], slots[s], gsems[s]
            )

        def drain(s):
            pltpu.make_async_copy(
                table_hbm.at[idx_v.at[pl.ds(0, _G)]], slots[s], gsems[s]
            ).wait()

        for s in range(_NSLOT):
            fire(s, s)

        def round_body(r, carry):
            for s in range(_NSLOT):
                c = r * _NSLOT + s
                drain(s)
                pltpu.sync_copy(slots[s], out_hbm.at[pl.ds(base + c * _G, _G)])
                fire(c + _NSLOT, s)
            return carry

        lax.fori_loop(0, n_rounds - 1, round_body, 0)

        for s in range(_NSLOT):
            c = (n_rounds - 1) * _NSLOT + s
            drain(s)
            pltpu.sync_copy(slots[s], out_hbm.at[pl.ds(base + c * _G, _G)])

    return gather_kernel


def kernel(x, m, W):
    b, f = x.shape
    total = b * f
    idx = x.reshape(total).astype(jnp.int32)
    out = _build(total)(idx, W)
    return out.reshape(b, f, _EMB_DIM)
